# baseline (speedup 1.0000x reference)
import functools
import jax
import jax.numpy as jnp
from jax import lax
from jax.experimental import pallas as pl
from jax.experimental.pallas import tpu as pltpu, tpu_sc as plsc

BATCH, SEQ, DM = 4, 2048, 2048
NC, NS, L = 2, 16, 16
NW = NC * NS
ROWS_PER_W = SEQ // NW
C = 4
NCH = ROWS_PER_W // C
VECS = DM // L


def _sc_body(in_hbm, t_hbm, out_hbm, t_v, x_v, sem_t, sem_x, sem_o):
    wid = lax.axis_index("s") * NC + lax.axis_index("c")
    s_base = wid * ROWS_PER_W

    def chunk(g):
        s0 = s_base + g * C
        pltpu.async_copy(t_hbm.at[pl.ds(s0, C), :], t_v, sem_t).wait()
        for b in range(BATCH):
            pltpu.async_copy(in_hbm.at[b, pl.ds(s0, C), :], x_v.at[b], sem_x).wait()

        def col(j):
            for r in range(C):
                t = t_v[r, pl.ds(j * L, L)]
                for b in range(BATCH):
                    x_v[b, r, pl.ds(j * L, L)] = x_v[b, r, pl.ds(j * L, L)] + t

        pl.loop(0, VECS)(col)

        for b in range(BATCH):
            pltpu.async_copy(x_v.at[b], out_hbm.at[b, pl.ds(s0, C), :], sem_o).wait()

    pl.loop(0, NCH)(chunk)


def kernel(inputs, pos_table):
    k = functools.partial(
        pl.kernel,
        out_type=jax.ShapeDtypeStruct((BATCH, SEQ, DM), jnp.float32),
        mesh=plsc.VectorSubcoreMesh(
            core_axis_name="c", subcore_axis_name="s", num_cores=NC, num_subcores=NS
        ),
        scratch_types=[
            pltpu.VMEM((C, DM), jnp.float32),
            pltpu.VMEM((BATCH, C, DM), jnp.float32),
            pltpu.SemaphoreType.DMA,
            pltpu.SemaphoreType.DMA,
            pltpu.SemaphoreType.DMA,
        ],
    )(_sc_body)
    return k(inputs, pos_table)
